# baseline (speedup 1.0000x reference)
import functools

import jax
import jax.numpy as jnp
from jax import lax
from jax.experimental import pallas as pl
from jax.experimental.pallas import tpu as pltpu
from jax.experimental.pallas import tpu_sc as plsc

NUM_JOKERS = 150
NUM_IDS = 178
ID_PAD = 192
NUM_ED = 5
NUM_SLOTS = 6
D = 128
TABLE_ROWS = ID_PAD * NUM_ED * NUM_SLOTS
ZERO_ROW = TABLE_ROWS - 1

NUM_WORKERS = 32
CHUNK = 64
TOK = CHUNK * NUM_SLOTS
GATHERS = TOK // 128


def _table_body(emb_ref, ed_ref, pos_ref, w_ref, b_ref, out_ref):
    i = pl.program_id(0)
    embb = emb_ref[...]
    edm = ed_ref[...] * (
        lax.broadcasted_iota(jnp.int32, (NUM_ED, D), 0) > 0
    ).astype(jnp.float32)
    pos = pos_ref[...]
    x = (embb[:, None, None, :] + edm[None, :, None, :] + pos[None, None, :, :])
    mean = jnp.mean(x, axis=-1, keepdims=True)
    var = jnp.mean((x - mean) ** 2, axis=-1, keepdims=True)
    y = (x - mean) * lax.rsqrt(var + 1e-5) * w_ref[...][None, None] + b_ref[...][None, None]
    ide = lax.broadcasted_iota(jnp.int32, x.shape, 0) + i * 8
    ee = lax.broadcasted_iota(jnp.int32, x.shape, 1)
    te = lax.broadcasted_iota(jnp.int32, x.shape, 2)
    is_zero_row = (ide == ID_PAD - 1) & (ee == NUM_ED - 1) & (te == NUM_SLOTS - 1)
    out_ref[...] = jnp.where(is_zero_row, 0.0, y)


def _build_table(emb_pad, ed, pos, w, b):
    return pl.pallas_call(
        _table_body,
        grid=(ID_PAD // 8,),
        in_specs=[
            pl.BlockSpec((8, D), lambda i: (i, 0)),
            pl.BlockSpec((NUM_ED, D), lambda i: (0, 0)),
            pl.BlockSpec((NUM_SLOTS, D), lambda i: (0, 0)),
            pl.BlockSpec((1, D), lambda i: (0, 0)),
            pl.BlockSpec((1, D), lambda i: (0, 0)),
        ],
        out_specs=pl.BlockSpec((8, NUM_ED, NUM_SLOTS, D), lambda i: (i, 0, 0, 0)),
        out_shape=jax.ShapeDtypeStruct((ID_PAD, NUM_ED, NUM_SLOTS, D), jnp.float32),
    )(emb_pad, ed, pos, w, b)


def _make_sc_kernel(batch):
    spw = batch // NUM_WORKERS
    nch = spw // CHUNK
    mesh = plsc.VectorSubcoreMesh(core_axis_name="c", subcore_axis_name="s")

    @functools.partial(
        pl.kernel,
        out_type=(
            jax.ShapeDtypeStruct((batch * NUM_SLOTS, D), jnp.float32),
            jax.ShapeDtypeStruct((batch * NUM_SLOTS,), jnp.int32),
        ),
        mesh=mesh,
        scratch_types=[
            pltpu.VMEM((CHUNK,), jnp.int32),
            pltpu.VMEM((CHUNK,), jnp.int32),
            pltpu.VMEM((CHUNK * 5,), jnp.int32),
            pltpu.VMEM((CHUNK * 5,), jnp.int32),
            pltpu.VMEM((CHUNK * 5,), jnp.int32),
            pltpu.VMEM((GATHERS, 128), jnp.int32),
            pltpu.VMEM((TOK,), jnp.int32),
            pltpu.VMEM((TOK, D), jnp.float32),
            pltpu.SemaphoreType.DMA,
        ],
        compiler_params=pltpu.CompilerParams(needs_layout_passes=False),
    )
    def sc_kernel(table, bs_h, act_h, jid_h, jm_h, jed_h, out_h, m_h,
                  bs_v, act_v, jid_v, jm_v, jed_v, idx_v, msk_v, gath_v, sem):
        wid = lax.axis_index("s") * 2 + lax.axis_index("c")

        def chunk_body(c, carry):
            s0 = wid * spw + c * CHUNK
            pltpu.sync_copy(bs_h.at[pl.ds(s0, CHUNK)], bs_v)
            pltpu.sync_copy(act_h.at[pl.ds(s0, CHUNK)], act_v)
            pltpu.sync_copy(jid_h.at[pl.ds(s0 * 5, CHUNK * 5)], jid_v)
            pltpu.sync_copy(jm_h.at[pl.ds(s0 * 5, CHUNK * 5)], jm_v)
            pltpu.sync_copy(jed_h.at[pl.ds(s0 * 5, CHUNK * 5)], jed_v)
            lane = lax.broadcasted_iota(jnp.int32, (16,), 0)
            zero = jnp.zeros((16,), jnp.int32)
            one = zero + 1
            for g in range(CHUNK // 16):
                bs16 = bs_v[pl.ds(g * 16, 16)]
                act16 = act_v[pl.ds(g * 16, 16)]
                hb = act16 != 0
                l5 = lane * 5 + g * 80
                jid = [plsc.load_gather(jid_v, [l5 + t]) for t in range(5)]
                jm = [plsc.load_gather(jm_v, [l5 + t]) for t in range(5)]
                jed = [plsc.load_gather(jed_v, [l5 + t]) for t in range(5)]
                anym = jm[0] | jm[1] | jm[2] | jm[3] | jm[4]
                nomod1 = jnp.where(anym == 0, one, zero)
                for t in range(NUM_SLOTS):
                    if t == 0:
                        idb, eb, mb = bs16 + NUM_JOKERS, zero, one
                    else:
                        idb, eb, mb = jid[t - 1], jed[t - 1], jm[t - 1]
                    if t < 5:
                        idn, en = jid[t], jed[t]
                        mn = (jm[0] | nomod1) if t == 0 else jm[t]
                    else:
                        idn, en, mn = zero, zero, zero
                    idv = jnp.where(hb, idb, idn)
                    ev = jnp.where(hb, eb, en)
                    mv = jnp.where(hb, mb, mn)
                    flat = idv * (NUM_ED * NUM_SLOTS) + ev * NUM_SLOTS + t
                    flat = jnp.where(mv != 0, flat, ZERO_ROW)
                    tok = lane * 6 + (g * 96 + t)
                    plsc.store_scatter(
                        idx_v,
                        [lax.shift_right_logical(tok, 7), lax.bitwise_and(tok, 127)],
                        flat,
                    )
                    plsc.store_scatter(msk_v, [tok], mv)
            cps = [
                pltpu.async_copy(
                    table.at[idx_v.at[k]], gath_v.at[pl.ds(k * 128, 128)], sem
                )
                for k in range(GATHERS)
            ]
            for cp in cps:
                cp.wait()
            t0 = s0 * NUM_SLOTS
            pltpu.sync_copy(gath_v, out_h.at[pl.ds(t0, TOK)])
            pltpu.sync_copy(msk_v, m_h.at[pl.ds(t0, TOK)])
            return carry

        lax.fori_loop(0, nch, chunk_body, 0)

    return sc_kernel


def kernel(boss_id, boss_is_active, joker_ids, joker_mask, joker_editions,
           emb, pos_emb, edition_emb, ln_w, ln_b):
    batch = boss_id.shape[0]
    emb_pad = jnp.zeros((ID_PAD, D), jnp.float32).at[:emb.shape[0]].set(emb)
    table4 = _build_table(
        emb_pad, edition_emb, pos_emb, ln_w.reshape(1, D), ln_b.reshape(1, D)
    )
    table = table4.reshape(TABLE_ROWS, D)
    sc = _make_sc_kernel(batch)
    seq, m32 = sc(
        table,
        boss_id.reshape(-1),
        boss_is_active.reshape(-1),
        joker_ids.reshape(-1),
        joker_mask.reshape(-1),
        joker_editions.reshape(-1),
    )
    return seq.reshape(batch, NUM_SLOTS, D), m32.reshape(batch, NUM_SLOTS).astype(bool)


# speedup vs baseline: 1.9233x; 1.9233x over previous
import functools

import jax
import jax.numpy as jnp
from jax import lax
from jax.experimental import pallas as pl
from jax.experimental.pallas import tpu as pltpu
from jax.experimental.pallas import tpu_sc as plsc

NUM_JOKERS = 150
NUM_IDS = 178
NUM_ED = 5
NUM_SLOTS = 6
D = 128
NUM_EDPOS = NUM_ED * NUM_SLOTS

NUM_WORKERS = 32
CHUNK = 64
TOK = CHUNK * NUM_SLOTS
PACK = 17

_F32_ONE_HALF = 0.5
_RSQRT_MAGIC = 0x5F3759DF


def _nrsqrt(v):
    i = plsc.bitcast(v, jnp.int32)
    i = _RSQRT_MAGIC - lax.shift_right_logical(i, 1)
    y = plsc.bitcast(i, jnp.float32)
    for _ in range(3):
        y = y * (1.5 - _F32_ONE_HALF * v * y * y)
    return y


def _make_sc_kernel(batch):
    spw = batch // NUM_WORKERS
    nch = spw // CHUNK
    mesh = plsc.VectorSubcoreMesh(core_axis_name="c", subcore_axis_name="s")

    @functools.partial(
        pl.kernel,
        out_type=(
            jax.ShapeDtypeStruct((batch * NUM_SLOTS, D), jnp.float32),
            jax.ShapeDtypeStruct((batch * NUM_SLOTS,), jnp.int32),
        ),
        mesh=mesh,
        scratch_types=[
            pltpu.VMEM((NUM_IDS * D,), jnp.float32),
            pltpu.VMEM((NUM_EDPOS * D,), jnp.float32),
            pltpu.VMEM((D,), jnp.float32),
            pltpu.VMEM((D,), jnp.float32),
            pltpu.VMEM((CHUNK * PACK,), jnp.int32),
            pltpu.VMEM((TOK,), jnp.int32),
            pltpu.VMEM((TOK,), jnp.int32),
            pltpu.VMEM((TOK,), jnp.int32),
            pltpu.VMEM((16 * D,), jnp.float32),
            pltpu.VMEM((TOK, D), jnp.float32),
            pltpu.SemaphoreType.DMA,
        ],
        compiler_params=pltpu.CompilerParams(needs_layout_passes=False),
    )
    def sc_kernel(emb_h, ed_h, pos_h, lnw_h, lnb_h, in_h, out_h, m_h,
                  emb_v, edpos_v, lnw_v, lnb_v, in_v, eid_v, epid_v, msk_v,
                  xbuf, out_v, sem):
        wid = lax.axis_index("s") * 2 + lax.axis_index("c")
        lane = lax.broadcasted_iota(jnp.int32, (16,), 0)
        zero = jnp.zeros((16,), jnp.int32)
        one = zero + 1

        pltpu.sync_copy(emb_h, emb_v)
        pltpu.sync_copy(lnw_h, lnw_v)
        pltpu.sync_copy(lnb_h, lnb_v)
        pltpu.sync_copy(ed_h, xbuf.at[pl.ds(0, NUM_ED * D)])
        pltpu.sync_copy(pos_h, xbuf.at[pl.ds(NUM_ED * D, NUM_SLOTS * D)])
        for e in range(NUM_ED):
            for t in range(NUM_SLOTS):
                for k in range(D // 16):
                    p = xbuf[pl.ds(NUM_ED * D + t * D + k * 16, 16)]
                    if e > 0:
                        p = p + xbuf[pl.ds(e * D + k * 16, 16)]
                    edpos_v[pl.ds((e * NUM_SLOTS + t) * D + k * 16, 16)] = p

        def chunk_body(c, carry):
            s0 = wid * spw + c * CHUNK
            pltpu.sync_copy(in_h.at[pl.ds(s0 * PACK, CHUNK * PACK)], in_v)

            def logic_body(g, carry2):
                l17 = lane * PACK + g * (16 * PACK)
                bs16 = plsc.load_gather(in_v, [l17])
                act16 = plsc.load_gather(in_v, [l17 + 1])
                hb = act16 != 0
                jid = [plsc.load_gather(in_v, [l17 + 2 + t]) for t in range(5)]
                jm = [plsc.load_gather(in_v, [l17 + 7 + t]) for t in range(5)]
                jed = [plsc.load_gather(in_v, [l17 + 12 + t]) for t in range(5)]
                anym = jm[0] | jm[1] | jm[2] | jm[3] | jm[4]
                nomod1 = jnp.where(anym == 0, one, zero)
                for t in range(NUM_SLOTS):
                    if t == 0:
                        idb, eb, mb = bs16 + NUM_JOKERS, zero, one
                    else:
                        idb, eb, mb = jid[t - 1], jed[t - 1], jm[t - 1]
                    if t < 5:
                        idn, en = jid[t], jed[t]
                        mn = (jm[0] | nomod1) if t == 0 else jm[t]
                    else:
                        idn, en, mn = zero, zero, zero
                    idv = jnp.where(hb, idb, idn)
                    ev = jnp.where(hb, eb, en)
                    mv = jnp.where(hb, mb, mn)
                    tok = lane * NUM_SLOTS + (g * 96 + t)
                    plsc.store_scatter(eid_v, [tok], idv)
                    plsc.store_scatter(epid_v, [tok], ev * NUM_SLOTS + t)
                    plsc.store_scatter(msk_v, [tok], mv)
                return carry2

            lax.fori_loop(0, CHUNK // 16, logic_body, 0)

            def tok_body(q, carry2):
                tokv = lane + q * 16
                ebase = eid_v[pl.ds(q * 16, 16)] * D
                pbase = epid_v[pl.ds(q * 16, 16)] * D
                mv = msk_v[pl.ds(q * 16, 16)]
                zf = jnp.zeros((16,), jnp.float32)

                def pass1(dd, acc):
                    s, sq = acc
                    for u in range(8):
                        d = dd * 8 + u
                        a = plsc.load_gather(emb_v, [ebase + d])
                        b = plsc.load_gather(edpos_v, [pbase + d])
                        x = a + b
                        s = s + x
                        sq = sq + x * x
                        xbuf[pl.ds(dd * 128 + u * 16, 16)] = x
                    return s, sq

                s, sq = lax.fori_loop(0, D // 8, pass1, (zf, zf))
                mean = s * (1.0 / D)
                var = sq * (1.0 / D) - mean * mean
                rs = _nrsqrt(var + 1e-5)
                mvf = mv.astype(jnp.float32)
                a_scale = rs * mvf
                a_shift = mean * a_scale

                def pass2(dd, carry3):
                    for u in range(8):
                        d = dd * 8 + u
                        x = xbuf[pl.ds(dd * 128 + u * 16, 16)]
                        wd = plsc.load_gather(lnw_v, [zero + d])
                        bd = plsc.load_gather(lnb_v, [zero + d])
                        t1 = x * a_scale - a_shift
                        y = t1 * wd + mvf * bd
                        plsc.store_scatter(out_v, [tokv, zero + d], y)
                    return carry3

                lax.fori_loop(0, D // 8, pass2, 0)
                return carry2

            lax.fori_loop(0, TOK // 16, tok_body, 0)

            t0 = s0 * NUM_SLOTS
            pltpu.sync_copy(out_v, out_h.at[pl.ds(t0, TOK)])
            pltpu.sync_copy(msk_v, m_h.at[pl.ds(t0, TOK)])
            return carry

        lax.fori_loop(0, nch, chunk_body, 0)

    return sc_kernel


def kernel(boss_id, boss_is_active, joker_ids, joker_mask, joker_editions,
           emb, pos_emb, edition_emb, ln_w, ln_b):
    batch = boss_id.shape[0]
    packed = jnp.concatenate(
        [boss_id, boss_is_active, joker_ids, joker_mask, joker_editions], axis=1
    ).reshape(-1)
    sc = _make_sc_kernel(batch)
    seq, m32 = sc(
        emb.reshape(-1),
        edition_emb.reshape(-1),
        pos_emb.reshape(-1),
        ln_w,
        ln_b,
        packed,
    )
    return seq.reshape(batch, NUM_SLOTS, D), m32.reshape(batch, NUM_SLOTS).astype(bool)


# speedup vs baseline: 5.7705x; 3.0003x over previous
import functools

import jax
import jax.numpy as jnp
from jax import lax
from jax.experimental import pallas as pl
from jax.experimental.pallas import tpu as pltpu
from jax.experimental.pallas import tpu_sc as plsc

NUM_JOKERS = 150
NUM_IDS = 178
NUM_ED = 5
NUM_SLOTS = 6
D = 128
NUM_EDPOS = NUM_ED * NUM_SLOTS
NUM_STATS = NUM_IDS * NUM_EDPOS

NUM_WORKERS = 32
CHUNK = 64
TOK = CHUNK * NUM_SLOTS
PACK = 17


def _stats_body(emb_ref, ed_ref, pos_ref, w_ref, b_ref,
                embw_ref, edposw_ref, rst_ref, mrst_ref):
    emb = emb_ref[...]
    edm = ed_ref[...] * (
        lax.broadcasted_iota(jnp.int32, (NUM_ED, D), 0) > 0
    ).astype(jnp.float32)
    pos = pos_ref[...]
    edpos = jnp.concatenate([edm[e][None, :] + pos for e in range(NUM_ED)], axis=0)
    se = jnp.sum(emb, axis=1)
    qe = jnp.sum(emb * emb, axis=1)
    sp = jnp.sum(edpos, axis=1)
    qp = jnp.sum(edpos * edpos, axis=1)
    m = jax.lax.dot_general(emb, edpos, (((1,), (1,)), ((), ())),
                            preferred_element_type=jnp.float32)
    mean = (se[:, None] + sp[None, :]) * (1.0 / D)
    msq = (qe[:, None] + qp[None, :] + 2.0 * m) * (1.0 / D)
    var = msq - mean * mean
    rst = lax.rsqrt(var + 1e-5)
    rst_ref[...] = rst
    mrst_ref[...] = mean * rst
    embw_ref[...] = emb * w_ref[...]
    edposw_ref[...] = edpos * w_ref[...]
    del b_ref


def _build_stats(emb, ed, pos, w, b):
    full = lambda s: pl.BlockSpec(s, lambda: tuple(0 for _ in s))
    return pl.pallas_call(
        _stats_body,
        grid=(),
        in_specs=[full((NUM_IDS, D)), full((NUM_ED, D)), full((NUM_SLOTS, D)),
                  full((1, D)), full((1, D))],
        out_specs=(full((NUM_IDS, D)), full((NUM_EDPOS, D)),
                   full((NUM_IDS, NUM_EDPOS)), full((NUM_IDS, NUM_EDPOS))),
        out_shape=(
            jax.ShapeDtypeStruct((NUM_IDS, D), jnp.float32),
            jax.ShapeDtypeStruct((NUM_EDPOS, D), jnp.float32),
            jax.ShapeDtypeStruct((NUM_IDS, NUM_EDPOS), jnp.float32),
            jax.ShapeDtypeStruct((NUM_IDS, NUM_EDPOS), jnp.float32),
        ),
    )(emb, ed, pos, w, b)


def _make_sc_kernel(batch):
    spw = batch // NUM_WORKERS
    nch = spw // CHUNK
    mesh = plsc.VectorSubcoreMesh(core_axis_name="c", subcore_axis_name="s")

    @functools.partial(
        pl.kernel,
        out_type=(
            jax.ShapeDtypeStruct((batch * NUM_SLOTS, D), jnp.float32),
            jax.ShapeDtypeStruct((batch * NUM_SLOTS,), jnp.int32),
        ),
        mesh=mesh,
        scratch_types=[
            pltpu.VMEM((NUM_IDS * D,), jnp.float32),
            pltpu.VMEM((NUM_EDPOS * D,), jnp.float32),
            pltpu.VMEM((NUM_STATS,), jnp.float32),
            pltpu.VMEM((NUM_STATS,), jnp.float32),
            pltpu.VMEM((D,), jnp.float32),
            pltpu.VMEM((D,), jnp.float32),
            pltpu.VMEM((CHUNK * PACK,), jnp.int32),
            pltpu.VMEM((TOK,), jnp.int32),
            pltpu.VMEM((TOK,), jnp.int32),
            pltpu.VMEM((TOK,), jnp.int32),
            pltpu.VMEM((TOK, D), jnp.float32),
            pltpu.SemaphoreType.DMA,
        ],
        compiler_params=pltpu.CompilerParams(needs_layout_passes=False),
    )
    def sc_kernel(embw_h, edposw_h, rst_h, mrst_h, lnw_h, lnb_h, in_h,
                  out_h, m_h,
                  embw_v, edposw_v, rst_v, mrst_v, lnw_v, lnb_v, in_v,
                  eid_v, epid_v, msk_v, out_v, sem):
        wid = lax.axis_index("s") * 2 + lax.axis_index("c")
        lane = lax.broadcasted_iota(jnp.int32, (16,), 0)
        zero = jnp.zeros((16,), jnp.int32)
        one = zero + 1

        pltpu.sync_copy(embw_h, embw_v)
        pltpu.sync_copy(edposw_h, edposw_v)
        pltpu.sync_copy(rst_h, rst_v)
        pltpu.sync_copy(mrst_h, mrst_v)
        pltpu.sync_copy(lnw_h, lnw_v)
        pltpu.sync_copy(lnb_h, lnb_v)

        def chunk_body(c, carry):
            s0 = wid * spw + c * CHUNK
            pltpu.sync_copy(in_h.at[pl.ds(s0 * PACK, CHUNK * PACK)], in_v)

            def logic_body(g, carry2):
                l17 = lane * PACK + g * (16 * PACK)
                bs16 = plsc.load_gather(in_v, [l17])
                act16 = plsc.load_gather(in_v, [l17 + 1])
                hb = act16 != 0
                jid = [plsc.load_gather(in_v, [l17 + 2 + t]) for t in range(5)]
                jm = [plsc.load_gather(in_v, [l17 + 7 + t]) for t in range(5)]
                jed = [plsc.load_gather(in_v, [l17 + 12 + t]) for t in range(5)]
                anym = jm[0] | jm[1] | jm[2] | jm[3] | jm[4]
                nomod1 = jnp.where(anym == 0, one, zero)
                for t in range(NUM_SLOTS):
                    if t == 0:
                        idb, eb, mb = bs16 + NUM_JOKERS, zero, one
                    else:
                        idb, eb, mb = jid[t - 1], jed[t - 1], jm[t - 1]
                    if t < 5:
                        idn, en = jid[t], jed[t]
                        mn = (jm[0] | nomod1) if t == 0 else jm[t]
                    else:
                        idn, en, mn = zero, zero, zero
                    idv = jnp.where(hb, idb, idn)
                    ev = jnp.where(hb, eb, en)
                    mv = jnp.where(hb, mb, mn)
                    tok = lane * NUM_SLOTS + (g * 96 + t)
                    plsc.store_scatter(eid_v, [tok], idv)
                    plsc.store_scatter(epid_v, [tok], ev * NUM_SLOTS + t)
                    plsc.store_scatter(msk_v, [tok], mv)
                return carry2

            lax.fori_loop(0, CHUNK // 16, logic_body, 0)

            def tok_body(q, carry2):
                idv = eid_v[pl.ds(q * 16, 16)]
                rv = epid_v[pl.ds(q * 16, 16)]
                mv = msk_v[pl.ds(q * 16, 16)]
                sidx = idv * NUM_EDPOS + rv
                rs = plsc.load_gather(rst_v, [sidx])
                mrs = plsc.load_gather(mrst_v, [sidx])
                mvf = mv.astype(jnp.float32)
                rsp = rs * mvf
                mrsp = mrs * mvf
                ebase = idv * D
                pbase = rv * D
                eb_s = [ebase[t] for t in range(16)]
                pb_s = [pbase[t] for t in range(16)]
                rs_s = [rsp[t] for t in range(16)]
                mrs_s = [mrsp[t] for t in range(16)]
                mv_s = [mvf[t] for t in range(16)]
                for k in range(D // 16):
                    wk = lnw_v[pl.ds(k * 16, 16)]
                    bk = lnb_v[pl.ds(k * 16, 16)]
                    for t in range(16):
                        x1 = embw_v[pl.ds(eb_s[t] + k * 16, 16)]
                        x2 = edposw_v[pl.ds(pb_s[t] + k * 16, 16)]
                        t2 = wk * mrs_s[t] - bk * mv_s[t]
                        y = (x1 + x2) * rs_s[t] - t2
                        out_v[q * 16 + t, pl.ds(k * 16, 16)] = y
                return carry2

            lax.fori_loop(0, TOK // 16, tok_body, 0)

            t0 = s0 * NUM_SLOTS
            pltpu.sync_copy(out_v, out_h.at[pl.ds(t0, TOK)])
            pltpu.sync_copy(msk_v, m_h.at[pl.ds(t0, TOK)])
            return carry

        lax.fori_loop(0, nch, chunk_body, 0)

    return sc_kernel


def kernel(boss_id, boss_is_active, joker_ids, joker_mask, joker_editions,
           emb, pos_emb, edition_emb, ln_w, ln_b):
    batch = boss_id.shape[0]
    packed = jnp.concatenate(
        [boss_id, boss_is_active, joker_ids, joker_mask, joker_editions], axis=1
    ).reshape(-1)
    embw, edposw, rst, mrst = _build_stats(
        emb, edition_emb, pos_emb, ln_w.reshape(1, D), ln_b.reshape(1, D)
    )
    sc = _make_sc_kernel(batch)
    seq, m32 = sc(
        embw.reshape(-1),
        edposw.reshape(-1),
        rst.reshape(-1),
        mrst.reshape(-1),
        ln_w,
        ln_b,
        packed,
    )
    return seq.reshape(batch, NUM_SLOTS, D), m32.reshape(batch, NUM_SLOTS).astype(bool)


# speedup vs baseline: 8.4101x; 1.4574x over previous
import functools

import jax
import jax.numpy as jnp
from jax import lax
from jax.experimental import pallas as pl
from jax.experimental.pallas import tpu as pltpu
from jax.experimental.pallas import tpu_sc as plsc

NUM_JOKERS = 150
NUM_IDS = 178
NUM_ED = 5
NUM_SLOTS = 6
D = 128
NUM_EDPOS = NUM_ED * NUM_SLOTS
NUM_STATS = NUM_IDS * NUM_EDPOS

NUM_WORKERS = 32
CHUNK = 64
TOK = CHUNK * NUM_SLOTS
PACK = 17


def _stats_body(emb_ref, ed_ref, pos_ref, w_ref, b_ref,
                embw_ref, edposw_ref, rst_ref, mrst_ref):
    emb = emb_ref[...]
    edm = ed_ref[...] * (
        lax.broadcasted_iota(jnp.int32, (NUM_ED, D), 0) > 0
    ).astype(jnp.float32)
    pos = pos_ref[...]
    edpos = jnp.concatenate([edm[e][None, :] + pos for e in range(NUM_ED)], axis=0)
    se = jnp.sum(emb, axis=1)
    qe = jnp.sum(emb * emb, axis=1)
    sp = jnp.sum(edpos, axis=1)
    qp = jnp.sum(edpos * edpos, axis=1)
    m = jax.lax.dot_general(emb, edpos, (((1,), (1,)), ((), ())),
                            preferred_element_type=jnp.float32)
    mean = (se[:, None] + sp[None, :]) * (1.0 / D)
    msq = (qe[:, None] + qp[None, :] + 2.0 * m) * (1.0 / D)
    var = msq - mean * mean
    rst = lax.rsqrt(var + 1e-5)
    rst_ref[...] = rst
    mrst_ref[...] = mean * rst
    embw_ref[...] = emb * w_ref[...]
    edposw_ref[...] = edpos * w_ref[...]
    del b_ref


def _build_stats(emb, ed, pos, w, b):
    full = lambda s: pl.BlockSpec(s, lambda: tuple(0 for _ in s))
    return pl.pallas_call(
        _stats_body,
        grid=(),
        in_specs=[full((NUM_IDS, D)), full((NUM_ED, D)), full((NUM_SLOTS, D)),
                  full((1, D)), full((1, D))],
        out_specs=(full((NUM_IDS, D)), full((NUM_EDPOS, D)),
                   full((NUM_IDS, NUM_EDPOS)), full((NUM_IDS, NUM_EDPOS))),
        out_shape=(
            jax.ShapeDtypeStruct((NUM_IDS, D), jnp.float32),
            jax.ShapeDtypeStruct((NUM_EDPOS, D), jnp.float32),
            jax.ShapeDtypeStruct((NUM_IDS, NUM_EDPOS), jnp.float32),
            jax.ShapeDtypeStruct((NUM_IDS, NUM_EDPOS), jnp.float32),
        ),
    )(emb, ed, pos, w, b)


def _make_sc_kernel(batch):
    spw = batch // NUM_WORKERS
    nch = spw // CHUNK
    mesh = plsc.VectorSubcoreMesh(core_axis_name="c", subcore_axis_name="s")

    @functools.partial(
        pl.kernel,
        out_type=(
            jax.ShapeDtypeStruct((batch * NUM_SLOTS * D,), jnp.float32),
            jax.ShapeDtypeStruct((batch * NUM_SLOTS,), jnp.int32),
        ),
        mesh=mesh,
        scratch_types=[
            pltpu.VMEM((NUM_IDS * D,), jnp.float32),
            pltpu.VMEM((NUM_EDPOS * D,), jnp.float32),
            pltpu.VMEM((NUM_STATS,), jnp.float32),
            pltpu.VMEM((NUM_STATS,), jnp.float32),
            pltpu.VMEM((D,), jnp.float32),
            pltpu.VMEM((D,), jnp.float32),
            pltpu.VMEM((CHUNK * PACK,), jnp.int32),
            pltpu.VMEM((TOK,), jnp.int32),
            pltpu.VMEM((TOK,), jnp.int32),
            pltpu.VMEM((TOK,), jnp.int32),
            pltpu.VMEM((TOK * D,), jnp.float32),
            pltpu.SemaphoreType.DMA,
        ],
        compiler_params=pltpu.CompilerParams(needs_layout_passes=False),
    )
    def sc_kernel(embw_h, edposw_h, rst_h, mrst_h, lnw_h, lnb_h, in_h,
                  out_h, m_h,
                  embw_v, edposw_v, rst_v, mrst_v, lnw_v, lnb_v, in_v,
                  eid_v, epid_v, msk_v, out_v, sem):
        wid = lax.axis_index("s") * 2 + lax.axis_index("c")
        lane = lax.broadcasted_iota(jnp.int32, (16,), 0)
        zero = jnp.zeros((16,), jnp.int32)
        one = zero + 1

        pltpu.sync_copy(embw_h, embw_v)
        pltpu.sync_copy(edposw_h, edposw_v)
        pltpu.sync_copy(rst_h, rst_v)
        pltpu.sync_copy(mrst_h, mrst_v)
        pltpu.sync_copy(lnw_h, lnw_v)
        pltpu.sync_copy(lnb_h, lnb_v)

        def chunk_body(c, carry):
            s0 = wid * spw + c * CHUNK
            pltpu.sync_copy(in_h.at[pl.ds(s0 * PACK, CHUNK * PACK)], in_v)

            def logic_body(g):
                l17 = lane * PACK + g * (16 * PACK)
                bs16 = plsc.load_gather(in_v, [l17])
                act16 = plsc.load_gather(in_v, [l17 + 1])
                hb = act16 != 0
                jid = [plsc.load_gather(in_v, [l17 + 2 + t]) for t in range(5)]
                jm = [plsc.load_gather(in_v, [l17 + 7 + t]) for t in range(5)]
                jed = [plsc.load_gather(in_v, [l17 + 12 + t]) for t in range(5)]
                anym = jm[0] | jm[1] | jm[2] | jm[3] | jm[4]
                nomod1 = jnp.where(anym == 0, one, zero)
                for t in range(NUM_SLOTS):
                    if t == 0:
                        idb, eb, mb = bs16 + NUM_JOKERS, zero, one
                    else:
                        idb, eb, mb = jid[t - 1], jed[t - 1], jm[t - 1]
                    if t < 5:
                        idn, en = jid[t], jed[t]
                        mn = (jm[0] | nomod1) if t == 0 else jm[t]
                    else:
                        idn, en, mn = zero, zero, zero
                    idv = jnp.where(hb, idb, idn)
                    ev = jnp.where(hb, eb, en)
                    mv = jnp.where(hb, mb, mn)
                    tok = lane * NUM_SLOTS + (g * 96 + t)
                    plsc.store_scatter(eid_v, [tok], idv)
                    plsc.store_scatter(epid_v, [tok], ev * NUM_SLOTS + t)
                    plsc.store_scatter(msk_v, [tok], mv)

            plsc.parallel_loop(0, CHUNK // 16)(logic_body)

            def tok_body(q):
                idv = eid_v[pl.ds(q * 16, 16)]
                rv = epid_v[pl.ds(q * 16, 16)]
                mv = msk_v[pl.ds(q * 16, 16)]
                sidx = idv * NUM_EDPOS + rv
                rs = plsc.load_gather(rst_v, [sidx])
                mrs = plsc.load_gather(mrst_v, [sidx])
                mvf = mv.astype(jnp.float32)
                rsp = rs * mvf
                mrsp = mrs * mvf
                ebase = idv * D
                pbase = rv * D
                eb_s = [ebase[t] for t in range(16)]
                pb_s = [pbase[t] for t in range(16)]
                rs_s = [rsp[t] for t in range(16)]
                mrs_s = [mrsp[t] for t in range(16)]
                mv_s = [mvf[t] for t in range(16)]
                ob = q * (16 * D)
                for k in range(D // 16):
                    wk = lnw_v[pl.ds(k * 16, 16)]
                    bk = lnb_v[pl.ds(k * 16, 16)]
                    for tb in range(0, 16, 4):
                        tt = range(tb, tb + 4)
                        x1s = [embw_v[pl.ds(eb_s[t] + k * 16, 16)] for t in tt]
                        x2s = [edposw_v[pl.ds(pb_s[t] + k * 16, 16)] for t in tt]
                        t2s = [wk * mrs_s[t] - bk * mv_s[t] for t in tt]
                        ys = [(x1s[i] + x2s[i]) * rs_s[t] - t2s[i]
                              for i, t in enumerate(tt)]
                        for i, t in enumerate(tt):
                            out_v[pl.ds(ob + t * D + k * 16, 16)] = ys[i]

            plsc.parallel_loop(0, TOK // 16)(tok_body)

            t0 = s0 * NUM_SLOTS
            pltpu.sync_copy(out_v, out_h.at[pl.ds(t0 * D, TOK * D)])
            pltpu.sync_copy(msk_v, m_h.at[pl.ds(t0, TOK)])
            return carry

        lax.fori_loop(0, nch, chunk_body, 0)

    return sc_kernel


def kernel(boss_id, boss_is_active, joker_ids, joker_mask, joker_editions,
           emb, pos_emb, edition_emb, ln_w, ln_b):
    batch = boss_id.shape[0]
    packed = jnp.concatenate(
        [boss_id, boss_is_active, joker_ids, joker_mask, joker_editions], axis=1
    ).reshape(-1)
    embw, edposw, rst, mrst = _build_stats(
        emb, edition_emb, pos_emb, ln_w.reshape(1, D), ln_b.reshape(1, D)
    )
    sc = _make_sc_kernel(batch)
    seq, m32 = sc(
        embw.reshape(-1),
        edposw.reshape(-1),
        rst.reshape(-1),
        mrst.reshape(-1),
        ln_w,
        ln_b,
        packed,
    )
    return (seq.reshape(batch, NUM_SLOTS, D),
            m32.reshape(batch, NUM_SLOTS).astype(bool))


# speedup vs baseline: 10.3931x; 1.2358x over previous
import functools

import jax
import jax.numpy as jnp
from jax import lax
from jax.experimental import pallas as pl
from jax.experimental.pallas import tpu as pltpu
from jax.experimental.pallas import tpu_sc as plsc

NUM_JOKERS = 150
NUM_IDS = 178
NUM_ED = 5
NUM_SLOTS = 6
D = 128
NUM_EDPOS = NUM_ED * NUM_SLOTS
NUM_STATS = NUM_IDS * NUM_EDPOS

NUM_WORKERS = 32
CHUNK = 64
TOK = CHUNK * NUM_SLOTS
PACK = 17


def _stats_body(emb_ref, ed_ref, pos_ref, w_ref, b_ref,
                embw_ref, edposw_ref, rst_ref, mrst_ref):
    emb = emb_ref[...]
    edm = ed_ref[...] * (
        lax.broadcasted_iota(jnp.int32, (NUM_ED, D), 0) > 0
    ).astype(jnp.float32)
    pos = pos_ref[...]
    edpos = jnp.concatenate([edm[e][None, :] + pos for e in range(NUM_ED)], axis=0)
    se = jnp.sum(emb, axis=1)
    qe = jnp.sum(emb * emb, axis=1)
    sp = jnp.sum(edpos, axis=1)
    qp = jnp.sum(edpos * edpos, axis=1)
    m = jax.lax.dot_general(emb, edpos, (((1,), (1,)), ((), ())),
                            preferred_element_type=jnp.float32)
    mean = (se[:, None] + sp[None, :]) * (1.0 / D)
    msq = (qe[:, None] + qp[None, :] + 2.0 * m) * (1.0 / D)
    var = msq - mean * mean
    rst = lax.rsqrt(var + 1e-5)
    rst_ref[...] = rst
    mrst_ref[...] = mean * rst
    embw_ref[...] = emb * w_ref[...]
    edposw_ref[...] = edpos * w_ref[...]
    del b_ref


def _build_stats(emb, ed, pos, w, b):
    full = lambda s: pl.BlockSpec(s, lambda: tuple(0 for _ in s))
    return pl.pallas_call(
        _stats_body,
        grid=(),
        in_specs=[full((NUM_IDS, D)), full((NUM_ED, D)), full((NUM_SLOTS, D)),
                  full((1, D)), full((1, D))],
        out_specs=(full((NUM_IDS, D)), full((NUM_EDPOS, D)),
                   full((NUM_IDS, NUM_EDPOS)), full((NUM_IDS, NUM_EDPOS))),
        out_shape=(
            jax.ShapeDtypeStruct((NUM_IDS, D), jnp.float32),
            jax.ShapeDtypeStruct((NUM_EDPOS, D), jnp.float32),
            jax.ShapeDtypeStruct((NUM_IDS, NUM_EDPOS), jnp.float32),
            jax.ShapeDtypeStruct((NUM_IDS, NUM_EDPOS), jnp.float32),
        ),
    )(emb, ed, pos, w, b)


def _make_sc_kernel(batch):
    spw = batch // NUM_WORKERS
    nch = spw // CHUNK
    mesh = plsc.VectorSubcoreMesh(core_axis_name="c", subcore_axis_name="s")

    @functools.partial(
        pl.kernel,
        out_type=(
            jax.ShapeDtypeStruct((batch, NUM_SLOTS, D), jnp.float32),
            jax.ShapeDtypeStruct((batch * NUM_SLOTS,), jnp.int32),
        ),
        mesh=mesh,
        scratch_types=[
            pltpu.VMEM((NUM_IDS * D,), jnp.float32),
            pltpu.VMEM((NUM_EDPOS * D,), jnp.float32),
            pltpu.VMEM((NUM_STATS,), jnp.float32),
            pltpu.VMEM((NUM_STATS,), jnp.float32),
            pltpu.VMEM((D,), jnp.float32),
            pltpu.VMEM((D,), jnp.float32),
            pltpu.VMEM((CHUNK * PACK,), jnp.int32),
            pltpu.VMEM((TOK,), jnp.int32),
            pltpu.VMEM((TOK,), jnp.int32),
            pltpu.VMEM((TOK,), jnp.int32),
            pltpu.VMEM((TOK, D), jnp.float32),
            pltpu.SemaphoreType.DMA,
        ],
        compiler_params=pltpu.CompilerParams(needs_layout_passes=False),
    )
    def sc_kernel(embw_h, edposw_h, rst_h, mrst_h, lnw_h, lnb_h, in_h,
                  out_h, m_h,
                  embw_v, edposw_v, rst_v, mrst_v, lnw_v, lnb_v, in_v,
                  eid_v, epid_v, msk_v, out_v, sem):
        wid = lax.axis_index("s") * 2 + lax.axis_index("c")
        lane = lax.broadcasted_iota(jnp.int32, (16,), 0)
        zero = jnp.zeros((16,), jnp.int32)
        one = zero + 1

        pltpu.sync_copy(embw_h, embw_v)
        pltpu.sync_copy(edposw_h, edposw_v)
        pltpu.sync_copy(rst_h, rst_v)
        pltpu.sync_copy(mrst_h, mrst_v)
        pltpu.sync_copy(lnw_h, lnw_v)
        pltpu.sync_copy(lnb_h, lnb_v)

        def chunk_body(c, carry):
            s0 = wid * spw + c * CHUNK
            pltpu.sync_copy(in_h.at[pl.ds(s0 * PACK, CHUNK * PACK)], in_v)

            def logic_body(g):
                l17 = lane * PACK + g * (16 * PACK)
                bs16 = plsc.load_gather(in_v, [l17])
                act16 = plsc.load_gather(in_v, [l17 + 1])
                hb = act16 != 0
                jid = [plsc.load_gather(in_v, [l17 + 2 + t]) for t in range(5)]
                jm = [plsc.load_gather(in_v, [l17 + 7 + t]) for t in range(5)]
                jed = [plsc.load_gather(in_v, [l17 + 12 + t]) for t in range(5)]
                anym = jm[0] | jm[1] | jm[2] | jm[3] | jm[4]
                nomod1 = jnp.where(anym == 0, one, zero)
                for t in range(NUM_SLOTS):
                    if t == 0:
                        idb, eb, mb = bs16 + NUM_JOKERS, zero, one
                    else:
                        idb, eb, mb = jid[t - 1], jed[t - 1], jm[t - 1]
                    if t < 5:
                        idn, en = jid[t], jed[t]
                        mn = (jm[0] | nomod1) if t == 0 else jm[t]
                    else:
                        idn, en, mn = zero, zero, zero
                    idv = jnp.where(hb, idb, idn)
                    ev = jnp.where(hb, eb, en)
                    mv = jnp.where(hb, mb, mn)
                    tok = lane * NUM_SLOTS + (g * 96 + t)
                    plsc.store_scatter(eid_v, [tok], idv)
                    plsc.store_scatter(epid_v, [tok], ev * NUM_SLOTS + t)
                    plsc.store_scatter(msk_v, [tok], mv)

            plsc.parallel_loop(0, CHUNK // 16)(logic_body)

            def tok_body(q):
                idv = eid_v[pl.ds(q * 16, 16)]
                rv = epid_v[pl.ds(q * 16, 16)]
                mv = msk_v[pl.ds(q * 16, 16)]
                sidx = idv * NUM_EDPOS + rv
                rs = plsc.load_gather(rst_v, [sidx])
                mrs = plsc.load_gather(mrst_v, [sidx])
                mvf = mv.astype(jnp.float32)
                rsp = rs * mvf
                mrsp = mrs * mvf
                ebase = idv * D
                pbase = rv * D
                eb_s = [ebase[t] for t in range(16)]
                pb_s = [pbase[t] for t in range(16)]
                rs_s = [rsp[t] for t in range(16)]
                mrs_s = [mrsp[t] for t in range(16)]
                mv_s = [mvf[t] for t in range(16)]
                tokv = lane + q * 16
                sv = tokv // NUM_SLOTS
                tv = tokv - sv * NUM_SLOTS
                rowv = tv * CHUNK + sv
                row_s = [rowv[t] for t in range(16)]
                for k in range(D // 16):
                    wk = lnw_v[pl.ds(k * 16, 16)]
                    bk = lnb_v[pl.ds(k * 16, 16)]
                    for tb in range(0, 16, 4):
                        tt = range(tb, tb + 4)
                        x1s = [embw_v[pl.ds(eb_s[t] + k * 16, 16)] for t in tt]
                        x2s = [edposw_v[pl.ds(pb_s[t] + k * 16, 16)] for t in tt]
                        t2s = [wk * mrs_s[t] - bk * mv_s[t] for t in tt]
                        ys = [(x1s[i] + x2s[i]) * rs_s[t] - t2s[i]
                              for i, t in enumerate(tt)]
                        for i, t in enumerate(tt):
                            out_v[row_s[t], pl.ds(k * 16, 16)] = ys[i]

            plsc.parallel_loop(0, TOK // 16)(tok_body)

            t0 = s0 * NUM_SLOTS
            for t in range(NUM_SLOTS):
                pltpu.sync_copy(out_v.at[pl.ds(t * CHUNK, CHUNK)],
                                out_h.at[pl.ds(s0, CHUNK), t])
            pltpu.sync_copy(msk_v, m_h.at[pl.ds(t0, TOK)])
            return carry

        lax.fori_loop(0, nch, chunk_body, 0)

    return sc_kernel


def kernel(boss_id, boss_is_active, joker_ids, joker_mask, joker_editions,
           emb, pos_emb, edition_emb, ln_w, ln_b):
    batch = boss_id.shape[0]
    packed = jnp.concatenate(
        [boss_id, boss_is_active, joker_ids, joker_mask, joker_editions], axis=1
    ).reshape(-1)
    embw, edposw, rst, mrst = _build_stats(
        emb, edition_emb, pos_emb, ln_w.reshape(1, D), ln_b.reshape(1, D)
    )
    sc = _make_sc_kernel(batch)
    seq, m32 = sc(
        embw.reshape(-1),
        edposw.reshape(-1),
        rst.reshape(-1),
        mrst.reshape(-1),
        ln_w,
        ln_b,
        packed,
    )
    return seq, m32.reshape(batch, NUM_SLOTS).astype(bool)
